# baseline (speedup 1.0000x reference)
import functools

import jax
import jax.numpy as jnp
from jax import lax
from jax.experimental import pallas as pl
from jax.experimental.pallas import tpu as pltpu
from jax.experimental.pallas import tpu_sc as plsc

NUM_CORES = 2
NUM_SUBCORES = 16
NUM_WORKERS = NUM_CORES * NUM_SUBCORES
CHUNK = 512


def kernel(input_ids, table):
    seq, batch = input_ids.shape
    vocab, dim = table.shape
    n = seq * batch
    assert n % NUM_WORKERS == 0
    n_per_w = n // NUM_WORKERS
    assert n_per_w % CHUNK == 0
    n_chunks = n_per_w // CHUNK

    flat_ids = input_ids.reshape(n).astype(jnp.int32)

    mesh = plsc.VectorSubcoreMesh(
        core_axis_name="c", subcore_axis_name="s",
        num_cores=NUM_CORES, num_subcores=NUM_SUBCORES)

    @functools.partial(
        pl.kernel,
        mesh=mesh,
        out_type=jax.ShapeDtypeStruct((n, dim), jnp.float32),
        scratch_types=[
            pltpu.VMEM((n_per_w,), jnp.int32),
            pltpu.VMEM((CHUNK, dim), jnp.float32),
            pltpu.SemaphoreType.DMA,
        ],
        compiler_params=pltpu.CompilerParams(use_tc_tiling_on_sc=False),
    )
    def emb(ids_hbm, table_hbm, out_hbm, idx_v, rows_v, sem):
        wid = lax.axis_index("s") * NUM_CORES + lax.axis_index("c")
        base = wid * n_per_w
        pltpu.sync_copy(ids_hbm.at[pl.ds(base, n_per_w)], idx_v)

        def body(c, carry):
            idx_chunk = idx_v.at[pl.ds(c * CHUNK, CHUNK)]
            pltpu.async_copy(table_hbm.at[idx_chunk], rows_v, sem).wait()
            pltpu.sync_copy(rows_v, out_hbm.at[pl.ds(base + c * CHUNK, CHUNK)])
            return carry

        lax.fori_loop(0, n_chunks, body, 0)

    out = emb(flat_ids, table)
    return out.reshape(seq, batch, dim)


# speedup vs baseline: 1.0266x; 1.0266x over previous
import functools

import jax
import jax.numpy as jnp
from jax import lax
from jax.experimental import pallas as pl
from jax.experimental.pallas import tpu as pltpu
from jax.experimental.pallas import tpu_sc as plsc

NUM_CORES = 2
NUM_SUBCORES = 16
NUM_WORKERS = NUM_CORES * NUM_SUBCORES
CHUNK = 256
NB = 4
K = 2


def kernel(input_ids, table):
    seq, batch = input_ids.shape
    vocab, dim = table.shape
    n = seq * batch
    assert n % NUM_WORKERS == 0
    n_per_w = n // NUM_WORKERS
    assert n_per_w % CHUNK == 0
    n_chunks = n_per_w // CHUNK
    assert n_chunks % NB == 0 and n_chunks >= 2 * NB
    n_rings = n_chunks // NB

    flat_ids = input_ids.reshape(n).astype(jnp.int32)

    mesh = plsc.VectorSubcoreMesh(
        core_axis_name="c", subcore_axis_name="s",
        num_cores=NUM_CORES, num_subcores=NUM_SUBCORES)

    @functools.partial(
        pl.kernel,
        mesh=mesh,
        out_type=jax.ShapeDtypeStruct((n, dim), jnp.float32),
        scratch_types=(
            [pltpu.VMEM((n_per_w,), jnp.int32)]
            + [pltpu.VMEM((CHUNK, dim), jnp.float32) for _ in range(NB)]
            + [pltpu.SemaphoreType.DMA for _ in range(2 * NB)]
        ),
        compiler_params=pltpu.CompilerParams(use_tc_tiling_on_sc=False),
    )
    def emb(ids_hbm, table_hbm, out_hbm, idx_v, *rest):
        bufs = rest[:NB]
        gs = rest[NB:2 * NB]
        ss = rest[2 * NB:3 * NB]
        wid = lax.axis_index("s") * NUM_CORES + lax.axis_index("c")
        base = wid * n_per_w
        pltpu.sync_copy(ids_hbm.at[pl.ds(base, n_per_w)], idx_v)

        def start_gather(c, b):
            idx_chunk = idx_v.at[pl.ds(c * CHUNK, CHUNK)]
            pltpu.async_copy(table_hbm.at[idx_chunk], bufs[b], gs[b])

        def wait_gather(c, b):
            idx_chunk = idx_v.at[pl.ds(c * CHUNK, CHUNK)]
            pltpu.make_async_copy(table_hbm.at[idx_chunk], bufs[b], gs[b]).wait()

        def start_store(c, b):
            pltpu.async_copy(
                bufs[b], out_hbm.at[pl.ds(base + c * CHUNK, CHUNK)], ss[b])

        def wait_store(c, b):
            pltpu.make_async_copy(
                bufs[b], out_hbm.at[pl.ds(base + c * CHUNK, CHUNK)], ss[b]).wait()

        for b in range(NB):
            start_gather(b, b)
            d = b - K
            if d >= 0:
                wait_gather(d, d)
                start_store(d, d)

        def ring(o, carry):
            for b in range(NB):
                c = o * NB + b
                wait_store(c - NB, b)
                start_gather(c, b)
                d = c - K
                bd = (b - K) % NB
                wait_gather(d, bd)
                start_store(d, bd)
            return carry

        lax.fori_loop(1, n_rings, ring, 0)

        for i in range(K):
            d = n_chunks - K + i
            wait_gather(d, d % NB)
            start_store(d, d % NB)
        for i in range(NB):
            d = n_chunks - NB + i
            wait_store(d, d % NB)

    out = emb(flat_ids, table)
    return out.reshape(seq, batch, dim)
